# baseline (speedup 1.0000x reference)
import functools

import jax
import jax.numpy as jnp
from jax import lax
from jax.experimental import pallas as pl
from jax.experimental.pallas import tpu as pltpu
from jax.experimental.pallas import tpu_sc as plsc

N = 10000
E = 320000
D = 128
HID = 128
NUM_REPS = 512

NC = 2
NS = 16
N_PAD = 10240
E_PAD = 327680
CHUNK = 128
E_ROWS = E_PAD // CHUNK
ROWS_PER_WORKER = E_ROWS // (NC * NS)
ROWS_PER_TILE = E_ROWS // NS
GROUP = 16
NODES_PER_TILE = N_PAD // NS
BLK = 512
GRID = N_PAD // BLK
BLK2 = 400
GRID2 = N // BLK2

_SELU_ALPHA = 1.6732632423543772
_SELU_SCALE = 1.0507009873554805
_HIGHEST = jax.lax.Precision.HIGHEST

_MESH = plsc.VectorSubcoreMesh(
    core_axis_name="c", subcore_axis_name="s", num_cores=NC, num_subcores=NS)


@functools.partial(
    pl.kernel,
    out_type=pltpu.HBM((NC, 2, N_PAD), jnp.float32),
    mesh=_MESH,
    scratch_types=[
        pltpu.VMEM((ROWS_PER_WORKER, CHUNK), jnp.int32),
        pltpu.VMEM((ROWS_PER_WORKER, CHUNK), jnp.int32),
        pltpu.VMEM((CHUNK,), jnp.float32),
        pltpu.VMEM_SHARED((N_PAD,), jnp.float32),
        pltpu.VMEM_SHARED((N_PAD,), jnp.float32),
        pltpu.SemaphoreType.DMA,
    ],
)
def _deg_kernel(s2d, r2d, zeros_n, out, idx_s, idx_r, ones_b, hist_s, hist_r,
                sem):
    c = lax.axis_index("c")
    s = lax.axis_index("s")
    w = c * NS + s
    for i in range(CHUNK // 16):
        ones_b[pl.ds(i * 16, 16)] = jnp.ones((16,), jnp.float32)
    zsl = pl.ds(s * NODES_PER_TILE, NODES_PER_TILE)
    pltpu.sync_copy(zeros_n.at[zsl], hist_s.at[zsl])
    pltpu.sync_copy(zeros_n.at[zsl], hist_r.at[zsl])
    row0 = w * ROWS_PER_WORKER
    pltpu.sync_copy(s2d.at[pl.ds(row0, ROWS_PER_WORKER), :], idx_s)
    pltpu.sync_copy(r2d.at[pl.ds(row0, ROWS_PER_WORKER), :], idx_r)
    plsc.subcore_barrier()

    def body(j, carry):
        d1 = pltpu.async_copy(ones_b, hist_s.at[idx_s.at[j]], sem, add=True)
        d2 = pltpu.async_copy(ones_b, hist_r.at[idx_r.at[j]], sem, add=True)
        d1.wait()
        d2.wait()
        return carry

    lax.fori_loop(0, ROWS_PER_WORKER, body, 0)
    plsc.subcore_barrier()
    pltpu.sync_copy(hist_s.at[zsl], out.at[c, 0, zsl])
    pltpu.sync_copy(hist_r.at[zsl], out.at[c, 1, zsl])


@functools.partial(
    pl.kernel,
    out_type=pltpu.HBM((NC, N_PAD, D), jnp.float32),
    mesh=_MESH,
    scratch_types=[
        pltpu.VMEM((GROUP, CHUNK), jnp.int32),
        pltpu.VMEM((GROUP, CHUNK), jnp.int32),
        pltpu.VMEM((2, CHUNK, D), jnp.float32),
        pltpu.VMEM_SHARED((N_PAD, D), jnp.float32),
        pltpu.SemaphoreType.DMA((2,)),
        pltpu.SemaphoreType.DMA((2,)),
    ],
)
def _agg_kernel(hcat, soff, r2d, zeros2d, out, sidx, ridx, rows, agg, gsem,
                ssem):
    c = lax.axis_index("c")
    s = lax.axis_index("s")
    zsl = pl.ds(s * NODES_PER_TILE, NODES_PER_TILE)
    pltpu.sync_copy(zeros2d.at[zsl, :], agg.at[zsl, :])
    plsc.subcore_barrier()
    row0 = s * ROWS_PER_TILE

    def group(g, carry):
        gr = row0 + g * GROUP
        pltpu.sync_copy(soff.at[c, pl.ds(gr, GROUP), :], sidx)
        pltpu.sync_copy(r2d.at[pl.ds(gr, GROUP), :], ridx)
        pltpu.async_copy(hcat.at[sidx.at[0]], rows.at[0], gsem.at[0])

        def body(j, carry2):
            b = lax.rem(j, 2)
            nb = lax.rem(j + 1, 2)

            @pl.when(jnp.logical_and(j >= 1, j + 1 < GROUP))
            def _():
                pltpu.make_async_copy(rows.at[nb], agg.at[ridx.at[j]],
                                      ssem.at[nb]).wait()

            @pl.when(j + 1 < GROUP)
            def _():
                pltpu.async_copy(hcat.at[sidx.at[j + 1]], rows.at[nb],
                                 gsem.at[nb])

            pltpu.make_async_copy(hcat.at[sidx.at[j]], rows.at[b],
                                  gsem.at[b]).wait()
            pltpu.async_copy(rows.at[b], agg.at[ridx.at[j]], ssem.at[b],
                             add=True)
            return carry2

        lax.fori_loop(0, GROUP, body, 0)
        pltpu.make_async_copy(rows.at[0], agg.at[ridx.at[0]],
                              ssem.at[0]).wait()
        pltpu.make_async_copy(rows.at[1], agg.at[ridx.at[1]],
                              ssem.at[1]).wait()
        return carry

    lax.fori_loop(0, ROWS_PER_TILE // GROUP, group, 0)
    plsc.subcore_barrier()
    pltpu.sync_copy(agg.at[zsl, :], out.at[c, zsl, :])


def _mm_scale_body(xc_ref, w_ref, b_ref, degs_ref, out_ref):
    i = pl.program_id(0)
    z = jnp.dot(xc_ref[...], w_ref[...],
                preferred_element_type=jnp.float32) + b_ref[...]
    dsum = degs_ref[...][:, 0:1] + degs_ref[...][:, 1:2]
    ss = lax.rsqrt(jnp.maximum(dsum, 1.0))
    nid = (lax.rem(i, GRID) * BLK
           + lax.broadcasted_iota(jnp.int32, (BLK, 1), 0))
    ss = jnp.where(nid < N, ss, 0.0)
    out_ref[...] = z * ss


def _mm_scale(xc, w, b2, deg_s2):
    return pl.pallas_call(
        _mm_scale_body,
        grid=(2 * GRID,),
        in_specs=[
            pl.BlockSpec((BLK, D), lambda i: (i, 0)),
            pl.BlockSpec((D, HID), lambda i: (0, 0)),
            pl.BlockSpec((1, HID), lambda i: (0, 0)),
            pl.BlockSpec((BLK, 2), lambda i: (lax.rem(i, GRID), 0)),
        ],
        out_specs=pl.BlockSpec((BLK, HID), lambda i: (i, 0)),
        out_shape=jax.ShapeDtypeStruct((2 * N_PAD, HID), jnp.float32),
    )(xc, w, b2, deg_s2)


def _selu(x):
    return _SELU_SCALE * jnp.where(x > 0, x, _SELU_ALPHA * (jnp.exp(x) - 1.0))


def _rr(degr):
    return lax.rsqrt(jnp.maximum(degr[:, 0:1] + degr[:, 1:2], 1.0))


def _post1_body(agg1_ref, degr_ref, cs_ref):
    i = pl.program_id(0)
    n1 = _selu(agg1_ref[...] * _rr(degr_ref[...]))

    @pl.when(i == 0)
    def _():
        cs_ref[...] = jnp.zeros_like(cs_ref)

    cs_ref[...] += jnp.sum(n1, axis=0, keepdims=True)


def _post1(agg1, deg_r2):
    return pl.pallas_call(
        _post1_body,
        grid=(GRID2,),
        in_specs=[
            pl.BlockSpec((BLK2, HID), lambda i: (i, 0)),
            pl.BlockSpec((BLK2, 2), lambda i: (i, 0)),
        ],
        out_specs=pl.BlockSpec((1, HID), lambda i: (0, 0)),
        out_shape=jax.ShapeDtypeStruct((1, HID), jnp.float32),
    )(agg1, deg_r2)


def _post2_body(agg1_ref, agg2_ref, degr_ref, cs_ref, wb_ref, cen_ref,
                ones_ref, h_ref, rep_ref, l1_ref, l2_ref, loss_ref):
    i = pl.program_id(0)
    rr = _rr(degr_ref[...])
    n1 = _selu(agg1_ref[...] * rr)
    n2 = _selu(agg2_ref[...] * rr)
    summ = jax.nn.sigmoid(cs_ref[...] * (1.0 / N))
    v = lax.dot_general(summ, wb_ref[...], (((1,), (1,)), ((), ())))
    l1_ref[...] = lax.dot_general(n1, v, (((1,), (1,)), ((), ())))
    l2_ref[...] = lax.dot_general(n2, v, (((1,), (1,)), ((), ())))
    nrm = jnp.sqrt(jnp.sum(n1 * n1, axis=1, keepdims=True))
    h = n1 / jnp.maximum(nrm, 1e-12)
    h_ref[...] = h
    cen = cen_ref[...]
    hh = jnp.sum(h * h, axis=1, keepdims=True)
    cc = lax.dot_general(ones_ref[...], cen * cen,
                         (((1,), (1,)), ((), ())), precision=_HIGHEST)
    g = lax.dot_general(h, cen, (((1,), (1,)), ((), ())))
    d2 = hh + cc - 2.0 * g
    dists = jnp.sqrt(jnp.maximum(d2, 0.0) + 1e-12)
    mind = jnp.min(dists, axis=1, keepdims=True)
    ids = lax.broadcasted_iota(jnp.int32, (BLK2, NUM_REPS), 1)
    rep_ref[...] = jnp.min(jnp.where(dists <= mind, ids, NUM_REPS), axis=1,
                           keepdims=True)
    contrib = jnp.sum(mind, keepdims=True)

    @pl.when(i == 0)
    def _():
        loss_ref[...] = jnp.zeros_like(loss_ref)

    loss_ref[...] += contrib


def _post2(agg1, agg2, deg_r2, colsum, wb, centers, ones_row):
    return pl.pallas_call(
        _post2_body,
        grid=(GRID2,),
        in_specs=[
            pl.BlockSpec((BLK2, HID), lambda i: (i, 0)),
            pl.BlockSpec((BLK2, HID), lambda i: (i, 0)),
            pl.BlockSpec((BLK2, 2), lambda i: (i, 0)),
            pl.BlockSpec((1, HID), lambda i: (0, 0)),
            pl.BlockSpec((HID, HID), lambda i: (0, 0)),
            pl.BlockSpec((NUM_REPS, HID), lambda i: (0, 0)),
            pl.BlockSpec((1, HID), lambda i: (0, 0)),
        ],
        out_specs=[
            pl.BlockSpec((BLK2, HID), lambda i: (i, 0)),
            pl.BlockSpec((BLK2, 1), lambda i: (i, 0)),
            pl.BlockSpec((BLK2, 1), lambda i: (i, 0)),
            pl.BlockSpec((BLK2, 1), lambda i: (i, 0)),
            pl.BlockSpec((1, 1), lambda i: (0, 0)),
        ],
        out_shape=[
            jax.ShapeDtypeStruct((N, HID), jnp.float32),
            jax.ShapeDtypeStruct((N, 1), jnp.int32),
            jax.ShapeDtypeStruct((N, 1), jnp.float32),
            jax.ShapeDtypeStruct((N, 1), jnp.float32),
            jax.ShapeDtypeStruct((1, 1), jnp.float32),
        ],
    )(agg1, agg2, deg_r2, colsum, wb, centers, ones_row)


def kernel(x, c_x, senders, receivers, W, b, Wb, centers):
    npad = E_PAD - E
    pad_idx = (N + jnp.arange(npad, dtype=jnp.int32) % (N_PAD - N))
    s_pad = jnp.concatenate([senders, pad_idx])
    r_pad = jnp.concatenate([receivers, pad_idx])
    s2d = s_pad.reshape(E_ROWS, CHUNK)
    r2d = r_pad.reshape(E_ROWS, CHUNK)
    soff = jnp.stack([s2d, s2d + N_PAD])

    zeros_n = jnp.zeros((N_PAD,), jnp.float32)
    zeros2d = jnp.zeros((N_PAD, D), jnp.float32)

    deg = _deg_kernel(s2d, r2d, zeros_n)
    deg_s2 = deg[:, 0, :].T
    deg_r2 = deg[:, 1, :].T

    xc = jnp.concatenate([
        jnp.pad(x, ((0, N_PAD - N), (0, 0))),
        jnp.pad(c_x, ((0, N_PAD - N), (0, 0))),
    ])
    hcat = _mm_scale(xc, W, b.reshape(1, HID), deg_s2)

    agg = _agg_kernel(hcat, soff, r2d, zeros2d)

    colsum = _post1(agg[0], deg_r2)
    ones_row = jnp.ones((1, HID), jnp.float32)
    h, rep, l1, l2, loss = _post2(agg[0], agg[1], deg_r2, colsum, Wb, centers,
                                  ones_row)

    rep_ids = rep[:, 0]
    logits = jnp.concatenate([l1[:, 0], l2[:, 0]])
    cluster_loss = loss[0, 0]
    return (h, centers, rep_ids, cluster_loss, logits)


# speedup vs baseline: 1.0783x; 1.0783x over previous
import functools

import jax
import jax.numpy as jnp
from jax import lax
from jax.experimental import pallas as pl
from jax.experimental.pallas import tpu as pltpu
from jax.experimental.pallas import tpu_sc as plsc

N = 10000
E = 320000
D = 128
HID = 128
NUM_REPS = 512

NC = 2
NS = 16
N_PAD = 10240
E_PAD = 327680
CHUNK = 128
E_ROWS = E_PAD // CHUNK
ROWS_PER_WORKER = E_ROWS // (NC * NS)
ACH = 64
A_ROWS = E_PAD // ACH
A_PER_TILE = A_ROWS // NS
AGRP = 16
NGRP = A_PER_TILE // AGRP
NBUF = 4
AHEAD = NBUF - 1
NODES_PER_TILE = N_PAD // NS
BLK = 512
GRID = N_PAD // BLK
BLK2 = 400
GRID2 = N // BLK2

_SELU_ALPHA = 1.6732632423543772
_SELU_SCALE = 1.0507009873554805
_HIGHEST = jax.lax.Precision.HIGHEST

_MESH = plsc.VectorSubcoreMesh(
    core_axis_name="c", subcore_axis_name="s", num_cores=NC, num_subcores=NS)


@functools.partial(
    pl.kernel,
    out_type=pltpu.HBM((NC, 2, N_PAD), jnp.float32),
    mesh=_MESH,
    scratch_types=[
        pltpu.VMEM((ROWS_PER_WORKER, CHUNK), jnp.int32),
        pltpu.VMEM((ROWS_PER_WORKER, CHUNK), jnp.int32),
        pltpu.VMEM((CHUNK,), jnp.float32),
        pltpu.VMEM_SHARED((N_PAD,), jnp.float32),
        pltpu.VMEM_SHARED((N_PAD,), jnp.float32),
        pltpu.SemaphoreType.DMA,
    ],
)
def _deg_kernel(s2d, r2d, zeros_n, out, idx_s, idx_r, ones_b, hist_s, hist_r,
                sem):
    c = lax.axis_index("c")
    s = lax.axis_index("s")
    w = c * NS + s
    for i in range(CHUNK // 16):
        ones_b[pl.ds(i * 16, 16)] = jnp.ones((16,), jnp.float32)
    zsl = pl.ds(s * NODES_PER_TILE, NODES_PER_TILE)
    pltpu.sync_copy(zeros_n.at[zsl], hist_s.at[zsl])
    pltpu.sync_copy(zeros_n.at[zsl], hist_r.at[zsl])
    row0 = w * ROWS_PER_WORKER
    pltpu.sync_copy(s2d.at[pl.ds(row0, ROWS_PER_WORKER), :], idx_s)
    pltpu.sync_copy(r2d.at[pl.ds(row0, ROWS_PER_WORKER), :], idx_r)
    plsc.subcore_barrier()

    def body(j, carry):
        d1 = pltpu.async_copy(ones_b, hist_s.at[idx_s.at[j]], sem, add=True)
        d2 = pltpu.async_copy(ones_b, hist_r.at[idx_r.at[j]], sem, add=True)
        d1.wait()
        d2.wait()
        return carry

    lax.fori_loop(0, ROWS_PER_WORKER, body, 0)
    plsc.subcore_barrier()
    pltpu.sync_copy(hist_s.at[zsl], out.at[c, 0, zsl])
    pltpu.sync_copy(hist_r.at[zsl], out.at[c, 1, zsl])


@functools.partial(
    pl.kernel,
    out_type=pltpu.HBM((NC, N_PAD, D), jnp.float32),
    mesh=_MESH,
    scratch_types=[
        pltpu.VMEM((3, AGRP, ACH), jnp.int32),
        pltpu.VMEM((3, AGRP, ACH), jnp.int32),
        pltpu.VMEM((NBUF, ACH, D), jnp.float32),
        pltpu.VMEM_SHARED((N_PAD, D), jnp.float32),
        pltpu.SemaphoreType.DMA((NBUF,)),
        pltpu.SemaphoreType.DMA((NBUF,)),
        pltpu.SemaphoreType.DMA((3,)),
    ],
)
def _agg_kernel(hcat, soff, r2d, zeros2d, out, sidx, ridx, rows, agg, gsem,
                ssem, isem):
    c = lax.axis_index("c")
    s = lax.axis_index("s")
    zsl = pl.ds(s * NODES_PER_TILE, NODES_PER_TILE)
    pltpu.sync_copy(zeros2d.at[zsl, :], agg.at[zsl, :])
    plsc.subcore_barrier()
    row0 = s * A_PER_TILE

    def idx_start(g, slot):
        gr = row0 + g * AGRP
        pltpu.async_copy(soff.at[c, pl.ds(gr, AGRP), :], sidx.at[slot],
                         isem.at[slot])
        pltpu.async_copy(r2d.at[pl.ds(gr, AGRP), :], ridx.at[slot],
                         isem.at[slot])

    def idx_wait(slot):
        pltpu.make_async_copy(soff.at[c, pl.ds(row0, AGRP), :],
                              sidx.at[slot], isem.at[slot]).wait()
        pltpu.make_async_copy(r2d.at[pl.ds(row0, AGRP), :],
                              ridx.at[slot], isem.at[slot]).wait()

    def gather_start(j):
        slot = lax.rem(lax.div(j, AGRP), 3)
        k = lax.rem(j, AGRP)
        b = lax.rem(j, NBUF)
        pltpu.async_copy(hcat.at[sidx.at[slot, k]], rows.at[b], gsem.at[b])

    idx_start(0, 0)
    idx_wait(0)
    idx_start(1, 1)
    for j in range(AHEAD):
        gather_start(j)

    def body(j, carry):
        b = lax.rem(j, NBUF)
        jn = j + AHEAD
        bn = lax.rem(jn, NBUF)

        @pl.when(jnp.logical_and(j >= 1, jn < A_PER_TILE))
        def _():
            pltpu.make_async_copy(rows.at[bn], agg.at[ridx.at[0, 0]],
                                  ssem.at[bn]).wait()

        @pl.when(lax.rem(j, AGRP) == 0)
        def _():
            g = lax.div(j, AGRP)

            @pl.when(g + 2 < NGRP)
            def _():
                idx_start(g + 2, lax.rem(g + 2, 3))

            @pl.when(g + 1 < NGRP)
            def _():
                idx_wait(lax.rem(g + 1, 3))

        @pl.when(jn < A_PER_TILE)
        def _():
            gather_start(jn)

        pltpu.make_async_copy(hcat.at[sidx.at[0, 0]], rows.at[b],
                              gsem.at[b]).wait()
        slot = lax.rem(lax.div(j, AGRP), 3)
        k = lax.rem(j, AGRP)
        pltpu.async_copy(rows.at[b], agg.at[ridx.at[slot, k]], ssem.at[b],
                         add=True)
        return carry

    lax.fori_loop(0, A_PER_TILE, body, 0)
    for b in range(NBUF):
        pltpu.make_async_copy(rows.at[b], agg.at[ridx.at[0, 0]],
                              ssem.at[b]).wait()
    plsc.subcore_barrier()
    pltpu.sync_copy(agg.at[zsl, :], out.at[c, zsl, :])


def _mm_scale_body(xc_ref, w_ref, b_ref, degs_ref, out_ref):
    i = pl.program_id(0)
    z = jnp.dot(xc_ref[...], w_ref[...],
                preferred_element_type=jnp.float32) + b_ref[...]
    dsum = degs_ref[...][:, 0:1] + degs_ref[...][:, 1:2]
    ss = lax.rsqrt(jnp.maximum(dsum, 1.0))
    nid = (lax.rem(i, GRID) * BLK
           + lax.broadcasted_iota(jnp.int32, (BLK, 1), 0))
    ss = jnp.where(nid < N, ss, 0.0)
    out_ref[...] = z * ss


def _mm_scale(xc, w, b2, deg_s2):
    return pl.pallas_call(
        _mm_scale_body,
        grid=(2 * GRID,),
        in_specs=[
            pl.BlockSpec((BLK, D), lambda i: (i, 0)),
            pl.BlockSpec((D, HID), lambda i: (0, 0)),
            pl.BlockSpec((1, HID), lambda i: (0, 0)),
            pl.BlockSpec((BLK, 2), lambda i: (lax.rem(i, GRID), 0)),
        ],
        out_specs=pl.BlockSpec((BLK, HID), lambda i: (i, 0)),
        out_shape=jax.ShapeDtypeStruct((2 * N_PAD, HID), jnp.float32),
    )(xc, w, b2, deg_s2)


def _selu(x):
    return _SELU_SCALE * jnp.where(x > 0, x, _SELU_ALPHA * (jnp.exp(x) - 1.0))


def _rr(degr):
    return lax.rsqrt(jnp.maximum(degr[:, 0:1] + degr[:, 1:2], 1.0))


def _post1_body(agg1_ref, degr_ref, cs_ref):
    i = pl.program_id(0)
    n1 = _selu(agg1_ref[...] * _rr(degr_ref[...]))

    @pl.when(i == 0)
    def _():
        cs_ref[...] = jnp.zeros_like(cs_ref)

    cs_ref[...] += jnp.sum(n1, axis=0, keepdims=True)


def _post1(agg1, deg_r2):
    return pl.pallas_call(
        _post1_body,
        grid=(GRID2,),
        in_specs=[
            pl.BlockSpec((BLK2, HID), lambda i: (i, 0)),
            pl.BlockSpec((BLK2, 2), lambda i: (i, 0)),
        ],
        out_specs=pl.BlockSpec((1, HID), lambda i: (0, 0)),
        out_shape=jax.ShapeDtypeStruct((1, HID), jnp.float32),
    )(agg1, deg_r2)


def _post2_body(agg1_ref, agg2_ref, degr_ref, cs_ref, wb_ref, cen_ref,
                ones_ref, h_ref, rep_ref, l1_ref, l2_ref, loss_ref):
    i = pl.program_id(0)
    rr = _rr(degr_ref[...])
    n1 = _selu(agg1_ref[...] * rr)
    n2 = _selu(agg2_ref[...] * rr)
    summ = jax.nn.sigmoid(cs_ref[...] * (1.0 / N))
    v = lax.dot_general(summ, wb_ref[...], (((1,), (1,)), ((), ())))
    l1_ref[...] = lax.dot_general(n1, v, (((1,), (1,)), ((), ())))
    l2_ref[...] = lax.dot_general(n2, v, (((1,), (1,)), ((), ())))
    nrm = jnp.sqrt(jnp.sum(n1 * n1, axis=1, keepdims=True))
    h = n1 / jnp.maximum(nrm, 1e-12)
    h_ref[...] = h
    cen = cen_ref[...]
    hh = jnp.sum(h * h, axis=1, keepdims=True)
    cc = lax.dot_general(ones_ref[...], cen * cen,
                         (((1,), (1,)), ((), ())), precision=_HIGHEST)
    g = lax.dot_general(h, cen, (((1,), (1,)), ((), ())))
    d2 = hh + cc - 2.0 * g
    dists = jnp.sqrt(jnp.maximum(d2, 0.0) + 1e-12)
    mind = jnp.min(dists, axis=1, keepdims=True)
    ids = lax.broadcasted_iota(jnp.int32, (BLK2, NUM_REPS), 1)
    rep_ref[...] = jnp.min(jnp.where(dists <= mind, ids, NUM_REPS), axis=1,
                           keepdims=True)
    contrib = jnp.sum(mind, keepdims=True)

    @pl.when(i == 0)
    def _():
        loss_ref[...] = jnp.zeros_like(loss_ref)

    loss_ref[...] += contrib


def _post2(agg1, agg2, deg_r2, colsum, wb, centers, ones_row):
    return pl.pallas_call(
        _post2_body,
        grid=(GRID2,),
        in_specs=[
            pl.BlockSpec((BLK2, HID), lambda i: (i, 0)),
            pl.BlockSpec((BLK2, HID), lambda i: (i, 0)),
            pl.BlockSpec((BLK2, 2), lambda i: (i, 0)),
            pl.BlockSpec((1, HID), lambda i: (0, 0)),
            pl.BlockSpec((HID, HID), lambda i: (0, 0)),
            pl.BlockSpec((NUM_REPS, HID), lambda i: (0, 0)),
            pl.BlockSpec((1, HID), lambda i: (0, 0)),
        ],
        out_specs=[
            pl.BlockSpec((BLK2, HID), lambda i: (i, 0)),
            pl.BlockSpec((BLK2, 1), lambda i: (i, 0)),
            pl.BlockSpec((BLK2, 1), lambda i: (i, 0)),
            pl.BlockSpec((BLK2, 1), lambda i: (i, 0)),
            pl.BlockSpec((1, 1), lambda i: (0, 0)),
        ],
        out_shape=[
            jax.ShapeDtypeStruct((N, HID), jnp.float32),
            jax.ShapeDtypeStruct((N, 1), jnp.int32),
            jax.ShapeDtypeStruct((N, 1), jnp.float32),
            jax.ShapeDtypeStruct((N, 1), jnp.float32),
            jax.ShapeDtypeStruct((1, 1), jnp.float32),
        ],
    )(agg1, agg2, deg_r2, colsum, wb, centers, ones_row)


def kernel(x, c_x, senders, receivers, W, b, Wb, centers):
    npad = E_PAD - E
    pad_idx = (N + jnp.arange(npad, dtype=jnp.int32) % (N_PAD - N))
    s_pad = jnp.concatenate([senders, pad_idx])
    r_pad = jnp.concatenate([receivers, pad_idx])
    s2d = s_pad.reshape(E_ROWS, CHUNK)
    r2d = r_pad.reshape(E_ROWS, CHUNK)
    ra2d = r_pad.reshape(A_ROWS, ACH)
    soff = jnp.stack([s_pad, s_pad + N_PAD]).reshape(2, A_ROWS, ACH)

    zeros_n = jnp.zeros((N_PAD,), jnp.float32)
    zeros2d = jnp.zeros((N_PAD, D), jnp.float32)

    deg = _deg_kernel(s2d, r2d, zeros_n)
    deg_s2 = deg[:, 0, :].T
    deg_r2 = deg[:, 1, :].T

    xc = jnp.concatenate([
        jnp.pad(x, ((0, N_PAD - N), (0, 0))),
        jnp.pad(c_x, ((0, N_PAD - N), (0, 0))),
    ])
    hcat = _mm_scale(xc, W, b.reshape(1, HID), deg_s2)

    agg = _agg_kernel(hcat, soff, ra2d, zeros2d)

    colsum = _post1(agg[0], deg_r2)
    ones_row = jnp.ones((1, HID), jnp.float32)
    h, rep, l1, l2, loss = _post2(agg[0], agg[1], deg_r2, colsum, Wb, centers,
                                  ones_row)

    rep_ids = rep[:, 0]
    logits = jnp.concatenate([l1[:, 0], l2[:, 0]])
    cluster_loss = loss[0, 0]
    return (h, centers, rep_ids, cluster_loss, logits)


# speedup vs baseline: 1.2391x; 1.1491x over previous
import functools

import jax
import jax.numpy as jnp
import numpy as np
from jax import lax
from jax.experimental import pallas as pl
from jax.experimental.pallas import tpu as pltpu
from jax.experimental.pallas import tpu_sc as plsc

N = 10000
E = 320000
D = 128
HID = 128
NUM_REPS = 512

NC = 2
NS = 16
N_PAD = 10240
E_PAD = 327680
CHUNK = 128
E_ROWS = E_PAD // CHUNK
ROWS_PER_WORKER = E_ROWS // (NC * NS)
ACH = 64
A_ROWS = E_PAD // ACH
A_PER_TILE = A_ROWS // NS
AGRP = 16
NGRP = A_PER_TILE // AGRP
NBUF = 4
AHEAD = NBUF - 1
NODES_PER_TILE = N_PAD // NS
BLK_MM = 1000
GRID_MM = N // BLK_MM
BLK1 = 2000
GRID1 = N // BLK1
BLK2 = 1000
GRID2 = N // BLK2
PADK = 4096

_SELU_ALPHA = 1.6732632423543772
_SELU_SCALE = 1.0507009873554805
_HIGHEST = jax.lax.Precision.HIGHEST

_MESH = plsc.VectorSubcoreMesh(
    core_axis_name="c", subcore_axis_name="s", num_cores=NC, num_subcores=NS)


@functools.partial(
    pl.kernel,
    out_type=pltpu.HBM((NC, 2, N_PAD), jnp.float32),
    mesh=_MESH,
    scratch_types=[
        pltpu.VMEM((ROWS_PER_WORKER, CHUNK), jnp.int32),
        pltpu.VMEM((ROWS_PER_WORKER, CHUNK), jnp.int32),
        pltpu.VMEM((CHUNK,), jnp.float32),
        pltpu.VMEM_SHARED((N_PAD,), jnp.float32),
        pltpu.VMEM_SHARED((N_PAD,), jnp.float32),
        pltpu.SemaphoreType.DMA,
    ],
)
def _deg_kernel(s2d, r2d, zeros_n, out, idx_s, idx_r, ones_b, hist_s, hist_r,
                sem):
    c = lax.axis_index("c")
    s = lax.axis_index("s")
    w = c * NS + s
    for i in range(CHUNK // 16):
        ones_b[pl.ds(i * 16, 16)] = jnp.ones((16,), jnp.float32)
    zsl = pl.ds(s * NODES_PER_TILE, NODES_PER_TILE)
    pltpu.sync_copy(zeros_n.at[zsl], hist_s.at[zsl])
    pltpu.sync_copy(zeros_n.at[zsl], hist_r.at[zsl])
    row0 = w * ROWS_PER_WORKER
    pltpu.sync_copy(s2d.at[pl.ds(row0, ROWS_PER_WORKER), :], idx_s)
    pltpu.sync_copy(r2d.at[pl.ds(row0, ROWS_PER_WORKER), :], idx_r)
    plsc.subcore_barrier()

    def body(j, carry):
        d1 = pltpu.async_copy(ones_b, hist_s.at[idx_s.at[j]], sem, add=True)
        d2 = pltpu.async_copy(ones_b, hist_r.at[idx_r.at[j]], sem, add=True)
        d1.wait()
        d2.wait()
        return carry

    lax.fori_loop(0, ROWS_PER_WORKER, body, 0)
    plsc.subcore_barrier()
    pltpu.sync_copy(hist_s.at[zsl], out.at[c, 0, zsl])
    pltpu.sync_copy(hist_r.at[zsl], out.at[c, 1, zsl])


@functools.partial(
    pl.kernel,
    out_type=pltpu.HBM((NC, N_PAD, D), jnp.float32),
    mesh=_MESH,
    scratch_types=[
        pltpu.VMEM((3, AGRP, ACH), jnp.int32),
        pltpu.VMEM((3, AGRP, ACH), jnp.int32),
        pltpu.VMEM((NBUF, ACH, D), jnp.float32),
        pltpu.VMEM_SHARED((N_PAD, D), jnp.float32),
        pltpu.SemaphoreType.DMA((NBUF,)),
        pltpu.SemaphoreType.DMA((NBUF,)),
        pltpu.SemaphoreType.DMA((3,)),
    ],
)
def _agg_kernel(hcat, soff, r2d, zeros2d, out, sidx, ridx, rows, agg, gsem,
                ssem, isem):
    c = lax.axis_index("c")
    s = lax.axis_index("s")
    zsl = pl.ds(s * NODES_PER_TILE, NODES_PER_TILE)
    pltpu.sync_copy(zeros2d.at[zsl, :], agg.at[zsl, :])
    plsc.subcore_barrier()
    row0 = s * A_PER_TILE

    def idx_start(g, slot):
        gr = row0 + g * AGRP
        pltpu.async_copy(soff.at[c, pl.ds(gr, AGRP), :], sidx.at[slot],
                         isem.at[slot])
        pltpu.async_copy(r2d.at[pl.ds(gr, AGRP), :], ridx.at[slot],
                         isem.at[slot])

    def idx_wait(slot):
        pltpu.make_async_copy(soff.at[c, pl.ds(row0, AGRP), :],
                              sidx.at[slot], isem.at[slot]).wait()
        pltpu.make_async_copy(r2d.at[pl.ds(row0, AGRP), :],
                              ridx.at[slot], isem.at[slot]).wait()

    def gather_start(j):
        slot = lax.rem(lax.div(j, AGRP), 3)
        k = lax.rem(j, AGRP)
        b = lax.rem(j, NBUF)
        pltpu.async_copy(hcat.at[sidx.at[slot, k]], rows.at[b], gsem.at[b])

    idx_start(0, 0)
    idx_wait(0)
    idx_start(1, 1)
    for j in range(AHEAD):
        gather_start(j)

    def body(j, carry):
        b = lax.rem(j, NBUF)
        jn = j + AHEAD
        bn = lax.rem(jn, NBUF)

        @pl.when(jnp.logical_and(j >= 1, jn < A_PER_TILE))
        def _():
            pltpu.make_async_copy(rows.at[bn], agg.at[ridx.at[0, 0]],
                                  ssem.at[bn]).wait()

        @pl.when(lax.rem(j, AGRP) == 0)
        def _():
            g = lax.div(j, AGRP)

            @pl.when(g + 2 < NGRP)
            def _():
                idx_start(g + 2, lax.rem(g + 2, 3))

            @pl.when(g + 1 < NGRP)
            def _():
                idx_wait(lax.rem(g + 1, 3))

        @pl.when(jn < A_PER_TILE)
        def _():
            gather_start(jn)

        pltpu.make_async_copy(hcat.at[sidx.at[0, 0]], rows.at[b],
                              gsem.at[b]).wait()
        slot = lax.rem(lax.div(j, AGRP), 3)
        k = lax.rem(j, AGRP)
        pltpu.async_copy(rows.at[b], agg.at[ridx.at[slot, k]], ssem.at[b],
                         add=True)
        return carry

    lax.fori_loop(0, A_PER_TILE, body, 0)
    for b in range(NBUF):
        pltpu.make_async_copy(rows.at[b], agg.at[ridx.at[0, 0]],
                              ssem.at[b]).wait()
    plsc.subcore_barrier()
    pltpu.sync_copy(agg.at[zsl, :], out.at[c, zsl, :])


def _mm_scale_body(x_ref, cx_ref, w_ref, b_ref, degs_ref, padc_ref, out_ref):
    w = w_ref[...]
    b = b_ref[...]
    z1 = jnp.dot(x_ref[...], w, preferred_element_type=jnp.float32) + b
    z2 = jnp.dot(cx_ref[...], w, preferred_element_type=jnp.float32) + b
    dsum = (degs_ref[...][:, 0:1] + degs_ref[...][:, 1:2]
            - padc_ref[...])
    ss = lax.rsqrt(jnp.maximum(dsum, 1.0))
    out_ref[0, :, :] = z1 * ss
    out_ref[1, :, :] = z2 * ss


def _mm_scale(x, c_x, w, b2, deg_s2, padc):
    return pl.pallas_call(
        _mm_scale_body,
        grid=(GRID_MM,),
        in_specs=[
            pl.BlockSpec((BLK_MM, D), lambda i: (i, 0)),
            pl.BlockSpec((BLK_MM, D), lambda i: (i, 0)),
            pl.BlockSpec((D, HID), lambda i: (0, 0)),
            pl.BlockSpec((1, HID), lambda i: (0, 0)),
            pl.BlockSpec((BLK_MM, 2), lambda i: (i, 0)),
            pl.BlockSpec((BLK_MM, 1), lambda i: (i, 0)),
        ],
        out_specs=pl.BlockSpec((2, BLK_MM, HID), lambda i: (0, i, 0)),
        out_shape=jax.ShapeDtypeStruct((2, N, HID), jnp.float32),
    )(x, c_x, w, b2, deg_s2, padc)


def _selu(x):
    return _SELU_SCALE * jnp.where(x > 0, x, _SELU_ALPHA * (jnp.exp(x) - 1.0))


def _rr(degr):
    return lax.rsqrt(jnp.maximum(degr[:, 0:1] + degr[:, 1:2], 1.0))


def _post1_body(agg1_ref, degr_ref, cs_ref):
    i = pl.program_id(0)
    n1 = _selu(agg1_ref[...] * _rr(degr_ref[...]))

    @pl.when(i == 0)
    def _():
        cs_ref[...] = jnp.zeros_like(cs_ref)

    cs_ref[...] += jnp.sum(n1, axis=0, keepdims=True)


def _post1(agg1, deg_r2):
    return pl.pallas_call(
        _post1_body,
        grid=(GRID1,),
        in_specs=[
            pl.BlockSpec((BLK1, HID), lambda i: (i, 0)),
            pl.BlockSpec((BLK1, 2), lambda i: (i, 0)),
        ],
        out_specs=pl.BlockSpec((1, HID), lambda i: (0, 0)),
        out_shape=jax.ShapeDtypeStruct((1, HID), jnp.float32),
    )(agg1, deg_r2)


def _post2_body(agg1_ref, agg2_ref, degr_ref, cs_ref, wb_ref, cen_ref,
                ones_ref, h_ref, rep_ref, l1_ref, l2_ref, loss_ref):
    i = pl.program_id(0)
    rr = _rr(degr_ref[...])
    n1 = _selu(agg1_ref[...] * rr)
    n2 = _selu(agg2_ref[...] * rr)
    summ = jax.nn.sigmoid(cs_ref[...] * (1.0 / N))
    v = lax.dot_general(summ, wb_ref[...], (((1,), (1,)), ((), ())))
    l1_ref[...] = lax.dot_general(n1, v, (((1,), (1,)), ((), ())))
    l2_ref[...] = lax.dot_general(n2, v, (((1,), (1,)), ((), ())))
    nrm = jnp.sqrt(jnp.sum(n1 * n1, axis=1, keepdims=True))
    h = n1 / jnp.maximum(nrm, 1e-12)
    h_ref[...] = h
    cen = cen_ref[...]
    hh = jnp.sum(h * h, axis=1, keepdims=True)
    cc = lax.dot_general(ones_ref[...], cen * cen,
                         (((1,), (1,)), ((), ())), precision=_HIGHEST)
    g = lax.dot_general(h, cen, (((1,), (1,)), ((), ())))
    d2 = hh + cc - 2.0 * g
    dists = jnp.sqrt(jnp.maximum(d2, 0.0) + 1e-12)
    mind = jnp.min(dists, axis=1, keepdims=True)
    ids = lax.broadcasted_iota(jnp.int32, (BLK2, NUM_REPS), 1)
    rep_ref[...] = jnp.min(jnp.where(dists <= mind, ids, NUM_REPS), axis=1,
                           keepdims=True)
    contrib = jnp.sum(mind, keepdims=True)

    @pl.when(i == 0)
    def _():
        loss_ref[...] = jnp.zeros_like(loss_ref)

    loss_ref[...] += contrib


def _post2(agg1, agg2, deg_r2, colsum, wb, centers, ones_row):
    return pl.pallas_call(
        _post2_body,
        grid=(GRID2,),
        in_specs=[
            pl.BlockSpec((BLK2, HID), lambda i: (i, 0)),
            pl.BlockSpec((BLK2, HID), lambda i: (i, 0)),
            pl.BlockSpec((BLK2, 2), lambda i: (i, 0)),
            pl.BlockSpec((1, HID), lambda i: (0, 0)),
            pl.BlockSpec((HID, HID), lambda i: (0, 0)),
            pl.BlockSpec((NUM_REPS, HID), lambda i: (0, 0)),
            pl.BlockSpec((1, HID), lambda i: (0, 0)),
        ],
        out_specs=[
            pl.BlockSpec((BLK2, HID), lambda i: (i, 0)),
            pl.BlockSpec((BLK2, 1), lambda i: (i, 0)),
            pl.BlockSpec((BLK2, 1), lambda i: (i, 0)),
            pl.BlockSpec((BLK2, 1), lambda i: (i, 0)),
            pl.BlockSpec((1, 1), lambda i: (0, 0)),
        ],
        out_shape=[
            jax.ShapeDtypeStruct((N, HID), jnp.float32),
            jax.ShapeDtypeStruct((N, 1), jnp.int32),
            jax.ShapeDtypeStruct((N, 1), jnp.float32),
            jax.ShapeDtypeStruct((N, 1), jnp.float32),
            jax.ShapeDtypeStruct((1, 1), jnp.float32),
        ],
    )(agg1, agg2, deg_r2, colsum, wb, centers, ones_row)


def kernel(x, c_x, senders, receivers, W, b, Wb, centers):
    npad = E_PAD - E
    pad_s = jnp.asarray(np.arange(npad, dtype=np.int32) % PADK)
    pad_r = jnp.asarray(N + np.arange(npad, dtype=np.int32) % (N_PAD - N))
    padc = jnp.asarray(
        np.bincount(np.arange(npad) % PADK, minlength=N)
        .astype(np.float32)[:, None])
    s_pad = jnp.concatenate([senders, pad_s])
    r_pad = jnp.concatenate([receivers, pad_r])
    s2d = s_pad.reshape(E_ROWS, CHUNK)
    r2d = r_pad.reshape(E_ROWS, CHUNK)
    ra2d = r_pad.reshape(A_ROWS, ACH)
    soff = jnp.stack([s_pad, s_pad + N]).reshape(2, A_ROWS, ACH)

    zeros_n = jnp.zeros((N_PAD,), jnp.float32)
    zeros2d = jnp.zeros((N_PAD, D), jnp.float32)

    deg = _deg_kernel(s2d, r2d, zeros_n)
    deg_s2 = deg[:, 0, :N].T
    deg_r2 = deg[:, 1, :N].T

    hcat = _mm_scale(x, c_x, W, b.reshape(1, HID), deg_s2, padc)
    hflat = hcat.reshape(2 * N, HID)

    agg = _agg_kernel(hflat, soff, ra2d, zeros2d)

    colsum = _post1(agg[0], deg_r2)
    ones_row = jnp.ones((1, HID), jnp.float32)
    h, rep, l1, l2, loss = _post2(agg[0], agg[1], deg_r2, colsum, Wb, centers,
                                  ones_row)

    rep_ids = rep[:, 0]
    logits = jnp.concatenate([l1[:, 0], l2[:, 0]])
    cluster_loss = loss[0, 0]
    return (h, centers, rep_ids, cluster_loss, logits)


# speedup vs baseline: 1.3460x; 1.0863x over previous
import functools

import jax
import jax.numpy as jnp
import numpy as np
from jax import lax
from jax.experimental import pallas as pl
from jax.experimental.pallas import tpu as pltpu
from jax.experimental.pallas import tpu_sc as plsc

N = 10000
E = 320000
D = 128
HID = 128
NUM_REPS = 512

NC = 2
NS = 16
N_PAD = 10240
E_PAD = 327680
CHUNK = 128
E_ROWS = E_PAD // CHUNK
ROWS_PER_WORKER = E_ROWS // (NC * NS)
ACH = 64
A_ROWS = E_PAD // ACH
A_PER_TILE = A_ROWS // NS
AGRP = 16
NGRP = A_PER_TILE // AGRP
NBUF = 4
AHEAD = NBUF - 1
NODES_PER_TILE = N_PAD // NS
BLK_MM = 1000
GRID_MM = N // BLK_MM
BLK1 = 2000
GRID1 = N // BLK1
BLK2 = 1000
GRID2 = N // BLK2
PADK = 4096

_SELU_ALPHA = 1.6732632423543772
_SELU_SCALE = 1.0507009873554805
_HIGHEST = jax.lax.Precision.HIGHEST

_MESH = plsc.VectorSubcoreMesh(
    core_axis_name="c", subcore_axis_name="s", num_cores=NC, num_subcores=NS)


@functools.partial(
    pl.kernel,
    out_type=pltpu.HBM((NC, 2, N_PAD), jnp.float32),
    mesh=_MESH,
    scratch_types=[
        pltpu.VMEM((ROWS_PER_WORKER, CHUNK), jnp.int32),
        pltpu.VMEM((ROWS_PER_WORKER, CHUNK), jnp.int32),
        pltpu.VMEM((CHUNK,), jnp.float32),
        pltpu.VMEM_SHARED((N_PAD,), jnp.float32),
        pltpu.VMEM_SHARED((N_PAD,), jnp.float32),
        pltpu.SemaphoreType.DMA,
    ],
)
def _deg_kernel(s2d, r2d, zeros_n, out, idx_s, idx_r, ones_b, hist_s, hist_r,
                sem):
    c = lax.axis_index("c")
    s = lax.axis_index("s")
    w = c * NS + s
    for i in range(CHUNK // 16):
        ones_b[pl.ds(i * 16, 16)] = jnp.ones((16,), jnp.float32)
    zsl = pl.ds(s * NODES_PER_TILE, NODES_PER_TILE)
    pltpu.sync_copy(zeros_n.at[zsl], hist_s.at[zsl])
    pltpu.sync_copy(zeros_n.at[zsl], hist_r.at[zsl])
    row0 = w * ROWS_PER_WORKER
    pltpu.sync_copy(s2d.at[pl.ds(row0, ROWS_PER_WORKER), :], idx_s)
    pltpu.sync_copy(r2d.at[pl.ds(row0, ROWS_PER_WORKER), :], idx_r)
    plsc.subcore_barrier()

    def body(j, carry):
        d1 = pltpu.async_copy(ones_b, hist_s.at[idx_s.at[j]], sem, add=True)
        d2 = pltpu.async_copy(ones_b, hist_r.at[idx_r.at[j]], sem, add=True)
        d1.wait()
        d2.wait()
        return carry

    lax.fori_loop(0, ROWS_PER_WORKER, body, 0)
    plsc.subcore_barrier()
    pltpu.sync_copy(hist_s.at[zsl], out.at[c, 0, zsl])
    pltpu.sync_copy(hist_r.at[zsl], out.at[c, 1, zsl])


@functools.partial(
    pl.kernel,
    out_type=pltpu.HBM((NC, N_PAD, D), jnp.float32),
    mesh=_MESH,
    scratch_types=[
        pltpu.VMEM((3, AGRP, ACH), jnp.int32),
        pltpu.VMEM((3, AGRP, ACH), jnp.int32),
        pltpu.VMEM((NBUF, ACH, D), jnp.float32),
        pltpu.VMEM_SHARED((N_PAD, D), jnp.float32),
        pltpu.SemaphoreType.DMA((NBUF,)),
        pltpu.SemaphoreType.DMA((NBUF,)),
        pltpu.SemaphoreType.DMA((3,)),
    ],
)
def _agg_kernel(hcat, s2a, r2d, zeros2d, out, sidx, ridx, rows, agg, gsem,
                ssem, isem):
    c = lax.axis_index("c")
    s = lax.axis_index("s")
    zsl = pl.ds(s * NODES_PER_TILE, NODES_PER_TILE)
    pltpu.sync_copy(zeros2d.at[zsl, :], agg.at[zsl, :])
    plsc.subcore_barrier()
    row0 = s * A_PER_TILE
    coff = (c * N) * jnp.ones((16,), jnp.int32)

    def idx_start(g, slot):
        gr = row0 + g * AGRP
        pltpu.async_copy(s2a.at[pl.ds(gr, AGRP), :], sidx.at[slot],
                         isem.at[slot])
        pltpu.async_copy(r2d.at[pl.ds(gr, AGRP), :], ridx.at[slot],
                         isem.at[slot])

    def idx_wait(slot):
        pltpu.make_async_copy(s2a.at[pl.ds(row0, AGRP), :],
                              sidx.at[slot], isem.at[slot]).wait()
        pltpu.make_async_copy(r2d.at[pl.ds(row0, AGRP), :],
                              ridx.at[slot], isem.at[slot]).wait()
        for k in range(AGRP):
            for q in range(ACH // 16):
                sl = pl.ds(q * 16, 16)
                sidx[slot, k, sl] = sidx[slot, k, sl] + coff

    def gather_start(j):
        slot = lax.rem(lax.div(j, AGRP), 3)
        k = lax.rem(j, AGRP)
        b = lax.rem(j, NBUF)
        pltpu.async_copy(hcat.at[sidx.at[slot, k]], rows.at[b], gsem.at[b])

    idx_start(0, 0)
    idx_wait(0)
    idx_start(1, 1)
    for j in range(AHEAD):
        gather_start(j)

    def body(j, carry):
        b = lax.rem(j, NBUF)
        jn = j + AHEAD
        bn = lax.rem(jn, NBUF)

        @pl.when(jnp.logical_and(j >= 1, jn < A_PER_TILE))
        def _():
            pltpu.make_async_copy(rows.at[bn], agg.at[ridx.at[0, 0]],
                                  ssem.at[bn]).wait()

        @pl.when(lax.rem(j, AGRP) == 0)
        def _():
            g = lax.div(j, AGRP)

            @pl.when(g + 2 < NGRP)
            def _():
                idx_start(g + 2, lax.rem(g + 2, 3))

            @pl.when(g + 1 < NGRP)
            def _():
                idx_wait(lax.rem(g + 1, 3))

        @pl.when(jn < A_PER_TILE)
        def _():
            gather_start(jn)

        pltpu.make_async_copy(hcat.at[sidx.at[0, 0]], rows.at[b],
                              gsem.at[b]).wait()
        slot = lax.rem(lax.div(j, AGRP), 3)
        k = lax.rem(j, AGRP)
        pltpu.async_copy(rows.at[b], agg.at[ridx.at[slot, k]], ssem.at[b],
                         add=True)
        return carry

    lax.fori_loop(0, A_PER_TILE, body, 0)
    for b in range(NBUF):
        pltpu.make_async_copy(rows.at[b], agg.at[ridx.at[0, 0]],
                              ssem.at[b]).wait()
    plsc.subcore_barrier()
    pltpu.sync_copy(agg.at[zsl, :], out.at[c, zsl, :])


def _mm_scale_body(x_ref, cx_ref, w_ref, b_ref, degs_ref, padc_ref, out_ref):
    w = w_ref[...]
    b = b_ref[...]
    z1 = jnp.dot(x_ref[...], w, preferred_element_type=jnp.float32) + b
    z2 = jnp.dot(cx_ref[...], w, preferred_element_type=jnp.float32) + b
    dsum = (degs_ref[...][:, 0:1] + degs_ref[...][:, 1:2]
            - padc_ref[...])
    ss = lax.rsqrt(jnp.maximum(dsum, 1.0))
    out_ref[0, :, :] = z1 * ss
    out_ref[1, :, :] = z2 * ss


def _mm_scale(x, c_x, w, b2, deg_s2, padc):
    return pl.pallas_call(
        _mm_scale_body,
        grid=(GRID_MM,),
        in_specs=[
            pl.BlockSpec((BLK_MM, D), lambda i: (i, 0)),
            pl.BlockSpec((BLK_MM, D), lambda i: (i, 0)),
            pl.BlockSpec((D, HID), lambda i: (0, 0)),
            pl.BlockSpec((1, HID), lambda i: (0, 0)),
            pl.BlockSpec((BLK_MM, 2), lambda i: (i, 0)),
            pl.BlockSpec((BLK_MM, 1), lambda i: (i, 0)),
        ],
        out_specs=pl.BlockSpec((2, BLK_MM, HID), lambda i: (0, i, 0)),
        out_shape=jax.ShapeDtypeStruct((2, N, HID), jnp.float32),
    )(x, c_x, w, b2, deg_s2, padc)


def _selu(x):
    return _SELU_SCALE * jnp.where(x > 0, x, _SELU_ALPHA * (jnp.exp(x) - 1.0))


def _rr(degr):
    return lax.rsqrt(jnp.maximum(degr[:, 0:1] + degr[:, 1:2], 1.0))


def _post1_body(agg1_ref, degr_ref, cs_ref):
    i = pl.program_id(0)
    n1 = _selu(agg1_ref[0] * _rr(degr_ref[...]))

    @pl.when(i == 0)
    def _():
        cs_ref[...] = jnp.zeros_like(cs_ref)

    cs_ref[...] += jnp.sum(n1, axis=0, keepdims=True)


def _post1(agg1, deg_r2):
    return pl.pallas_call(
        _post1_body,
        grid=(GRID1,),
        in_specs=[
            pl.BlockSpec((1, BLK1, HID), lambda i: (0, i, 0)),
            pl.BlockSpec((BLK1, 2), lambda i: (i, 0)),
        ],
        out_specs=pl.BlockSpec((1, HID), lambda i: (0, 0)),
        out_shape=jax.ShapeDtypeStruct((1, HID), jnp.float32),
    )(agg1, deg_r2)


def _post2_body(agg1_ref, agg2_ref, degr_ref, cs_ref, wb_ref, cen_ref,
                ones_ref, h_ref, rep_ref, l1_ref, l2_ref, loss_ref):
    i = pl.program_id(0)
    rr = _rr(degr_ref[...])
    n1 = _selu(agg1_ref[0] * rr)
    n2 = _selu(agg2_ref[0] * rr)
    summ = jax.nn.sigmoid(cs_ref[...] * (1.0 / N))
    v = lax.dot_general(summ, wb_ref[...], (((1,), (1,)), ((), ())))
    l1_ref[0] = lax.dot_general(v, n1, (((1,), (1,)), ((), ())))
    l2_ref[0] = lax.dot_general(v, n2, (((1,), (1,)), ((), ())))
    nrm = jnp.sqrt(jnp.sum(n1 * n1, axis=1, keepdims=True))
    h = n1 / jnp.maximum(nrm, 1e-12)
    h_ref[...] = h
    cen = cen_ref[...]
    hh = jnp.sum(h * h, axis=1, keepdims=True)
    cc = lax.dot_general(ones_ref[...], cen * cen,
                         (((1,), (1,)), ((), ())), precision=_HIGHEST)
    g = lax.dot_general(h, cen, (((1,), (1,)), ((), ())))
    d2 = hh + cc - 2.0 * g
    dists = jnp.sqrt(jnp.maximum(d2, 0.0) + 1e-12)
    mind = jnp.min(dists, axis=1, keepdims=True)
    ids = lax.broadcasted_iota(jnp.int32, (BLK2, NUM_REPS), 1)
    rep_ref[...] = jnp.min(jnp.where(dists <= mind, ids, NUM_REPS), axis=1,
                           keepdims=True)
    contrib = jnp.sum(mind, keepdims=True)

    @pl.when(i == 0)
    def _():
        loss_ref[...] = jnp.zeros_like(loss_ref)

    loss_ref[...] += contrib


def _post2(agg, deg_r2, colsum, wb, centers, ones_row):
    return pl.pallas_call(
        _post2_body,
        grid=(GRID2,),
        in_specs=[
            pl.BlockSpec((1, BLK2, HID), lambda i: (0, i, 0)),
            pl.BlockSpec((1, BLK2, HID), lambda i: (1, i, 0)),
            pl.BlockSpec((BLK2, 2), lambda i: (i, 0)),
            pl.BlockSpec((1, HID), lambda i: (0, 0)),
            pl.BlockSpec((HID, HID), lambda i: (0, 0)),
            pl.BlockSpec((NUM_REPS, HID), lambda i: (0, 0)),
            pl.BlockSpec((1, HID), lambda i: (0, 0)),
        ],
        out_specs=[
            pl.BlockSpec((BLK2, HID), lambda i: (i, 0)),
            pl.BlockSpec((BLK2, 1), lambda i: (i, 0)),
            pl.BlockSpec((1, 1, BLK2), lambda i: (i, 0, 0)),
            pl.BlockSpec((1, 1, BLK2), lambda i: (i, 0, 0)),
            pl.BlockSpec((1, 1), lambda i: (0, 0)),
        ],
        out_shape=[
            jax.ShapeDtypeStruct((N, HID), jnp.float32),
            jax.ShapeDtypeStruct((N, 1), jnp.int32),
            jax.ShapeDtypeStruct((GRID2, 1, BLK2), jnp.float32),
            jax.ShapeDtypeStruct((GRID2, 1, BLK2), jnp.float32),
            jax.ShapeDtypeStruct((1, 1), jnp.float32),
        ],
    )(agg, agg, deg_r2, colsum, wb, centers, ones_row)


def kernel(x, c_x, senders, receivers, W, b, Wb, centers):
    npad = E_PAD - E
    pad_s = jnp.asarray(np.arange(npad, dtype=np.int32) % PADK)
    pad_r = jnp.asarray(N + np.arange(npad, dtype=np.int32) % (N_PAD - N))
    padc = jnp.asarray(
        np.bincount(np.arange(npad) % PADK, minlength=N)
        .astype(np.float32)[:, None])
    s_pad = jnp.concatenate([senders, pad_s])
    r_pad = jnp.concatenate([receivers, pad_r])
    s2d = s_pad.reshape(E_ROWS, CHUNK)
    r2d = r_pad.reshape(E_ROWS, CHUNK)
    sa2d = s_pad.reshape(A_ROWS, ACH)
    ra2d = r_pad.reshape(A_ROWS, ACH)

    zeros_n = jnp.zeros((N_PAD,), jnp.float32)
    zeros2d = jnp.zeros((N_PAD, D), jnp.float32)

    deg = _deg_kernel(s2d, r2d, zeros_n)
    deg_s2 = deg[:, 0, :N].T
    deg_r2 = deg[:, 1, :N].T

    hcat = _mm_scale(x, c_x, W, b.reshape(1, HID), deg_s2, padc)
    hflat = hcat.reshape(2 * N, HID)

    agg = _agg_kernel(hflat, sa2d, ra2d, zeros2d)

    colsum = _post1(agg, deg_r2)
    ones_row = jnp.ones((1, HID), jnp.float32)
    h, rep, l1, l2, loss = _post2(agg, deg_r2, colsum, Wb, centers, ones_row)

    rep_ids = rep[:, 0]
    logits = jnp.concatenate([l1.reshape(-1), l2.reshape(-1)])
    cluster_loss = loss[0, 0]
    return (h, centers, rep_ids, cluster_loss, logits)
